# baseline (speedup 1.0000x reference)
import functools

import jax
import jax.numpy as jnp
from jax import lax
from jax.experimental import pallas as pl
from jax.experimental.pallas import tpu as pltpu
from jax.experimental.pallas import tpu_sc as plsc

LANES = 16


def _sc_lookup(B, L, N, H):
    info = plsc.get_sparse_core_info()
    NC, NS = info.num_cores, info.num_subcores
    NW = NC * NS
    P = B * L
    assert P % NW == 0
    per_w = P // NW
    CH = 80
    NBUF = 2
    assert per_w % (CH * NBUF) == 0 and CH % 8 == 0 and CH % LANES == 0
    n_iters = per_w // (CH * NBUF)
    groups = per_w // LANES
    gpw = (CH * NBUF) // LANES
    mesh = plsc.VectorSubcoreMesh(core_axis_name="c", subcore_axis_name="s")

    @functools.partial(
        pl.kernel,
        out_type=jax.ShapeDtypeStruct((P, H), jnp.float32),
        mesh=mesh,
        scratch_types=[
            pltpu.VMEM((per_w,), jnp.int32),
            pltpu.VMEM((per_w,), jnp.int32),
            pltpu.VMEM((3, H), jnp.float32),
            [pltpu.VMEM((CH, H), jnp.float32) for _ in range(NBUF)],
            [pltpu.SemaphoreType.DMA for _ in range(NBUF)],
            [pltpu.SemaphoreType.DMA for _ in range(NBUF)],
            pltpu.SemaphoreType.DMA,
        ],
    )
    def k(tok_hbm, table_hbm, spec_hbm, out_hbm,
          tok_v, idx_v, sp3, ebufs, gsems, wsems, fsem):
        wid = lax.axis_index("s") * NC + lax.axis_index("c")
        base = wid * per_w

        pltpu.sync_copy(tok_hbm.at[pl.ds(base, per_w)], tok_v)
        pltpu.sync_copy(spec_hbm, sp3)

        def compute_idx(g0, g1):
            def idx_body(g, n):
                off = pl.ds(g * LANES, LANES)
                t = tok_v[off]
                p = base + g * LANES + lax.iota(jnp.int32, LANES)
                b = lax.div(p, L)
                e = jnp.clip(t - 3, 0, N * N - 1)
                idx_v[off] = b * (N * N) + e
                return n
            lax.fori_loop(g0, g1, idx_body, 0)
        compute_idx(0, gpw)

        def gather_of(c, s):
            return (table_hbm.at[idx_v.at[pl.ds(c * CH, CH)]],
                    ebufs[s], gsems[s])

        def write_of(c, s):
            return (ebufs[s],
                    out_hbm.at[pl.ds(base + c * CH, CH)], wsems[s])

        def scan_specials(g0, g1, nfix):
            def scan_body(g, n):
                v = tok_v[pl.ds(g * LANES, LANES)]
                for lane in range(LANES):
                    t = v[lane]
                    @pl.when(t < 3)
                    def _():
                        srow = jnp.clip(t, 0, 2)
                        pltpu.async_copy(
                            sp3.at[pl.ds(srow, 1)],
                            out_hbm.at[pl.ds(base + g * LANES + lane, 1)],
                            fsem)
                    n = n + jnp.where(t < 3, 1, 0)
                return n
            return lax.fori_loop(g0, g1, scan_body, nfix)

        def pipe_body(i, nfix):
            for s in range(NBUF):
                c = i * NBUF + s
                @pl.when(i > 0)
                def _():
                    pltpu.make_async_copy(*write_of(c - NBUF, s)).wait()
                pltpu.async_copy(*gather_of(c, s))
            compute_idx((i + 1) * gpw, jnp.minimum(i + 2, n_iters) * gpw)
            nfix = scan_specials(
                jnp.maximum(i - 1, 0) * gpw, i * gpw, nfix)
            for s in range(NBUF):
                c = i * NBUF + s
                pltpu.make_async_copy(*gather_of(c, s)).wait()
                pltpu.async_copy(*write_of(c, s))
            return nfix
        nfix = lax.fori_loop(0, n_iters, pipe_body, jnp.int32(0))
        for s in range(NBUF):
            c = (n_iters - 1) * NBUF + s
            pltpu.make_async_copy(*write_of(c, s)).wait()
        nfix = scan_specials((n_iters - 1) * gpw, n_iters * gpw, nfix)

        def drain_body(j, z):
            pltpu.make_async_copy(
                sp3.at[pl.ds(0, 1)], out_hbm.at[pl.ds(base, 1)], fsem).wait()
            return z
        lax.fori_loop(0, nfix, drain_body, 0)

    return k


def kernel(tokens, edge_emb, special_token_emb, token_to_edge):
    B, L = tokens.shape
    _, N, _, H = edge_emb.shape
    del token_to_edge
    tok = tokens.reshape(B * L)
    table = edge_emb.reshape(B * N * N, H)
    out = _sc_lookup(B, L, N, H)(tok, table, special_token_emb)
    return out.reshape(B, L, H)


# speedup vs baseline: 1.0467x; 1.0467x over previous
import functools

import jax
import jax.numpy as jnp
from jax import lax
from jax.experimental import pallas as pl
from jax.experimental.pallas import tpu as pltpu
from jax.experimental.pallas import tpu_sc as plsc

LANES = 16


def _sc_lookup(B, L, N, H):
    info = plsc.get_sparse_core_info()
    NC, NS = info.num_cores, info.num_subcores
    NW = NC * NS
    P = B * L
    assert P % NW == 0
    per_w = P // NW
    CH = 16
    NBUF = 10
    assert per_w % (CH * NBUF) == 0 and CH % 8 == 0 and CH % LANES == 0
    n_iters = per_w // (CH * NBUF)
    groups = per_w // LANES
    gpw = (CH * NBUF) // LANES
    mesh = plsc.VectorSubcoreMesh(core_axis_name="c", subcore_axis_name="s")

    @functools.partial(
        pl.kernel,
        out_type=jax.ShapeDtypeStruct((P, H), jnp.float32),
        mesh=mesh,
        scratch_types=[
            pltpu.VMEM((per_w,), jnp.int32),
            pltpu.VMEM((per_w,), jnp.int32),
            pltpu.VMEM((3, H), jnp.float32),
            [pltpu.VMEM((CH, H), jnp.float32) for _ in range(NBUF)],
            [pltpu.SemaphoreType.DMA for _ in range(NBUF)],
            [pltpu.SemaphoreType.DMA for _ in range(NBUF)],
            pltpu.SemaphoreType.DMA,
        ],
    )
    def k(tok_hbm, table_hbm, spec_hbm, out_hbm,
          tok_v, idx_v, sp3, ebufs, gsems, wsems, fsem):
        wid = lax.axis_index("s") * NC + lax.axis_index("c")
        base = wid * per_w

        pltpu.sync_copy(tok_hbm.at[pl.ds(base, per_w)], tok_v)
        pltpu.sync_copy(spec_hbm, sp3)

        def compute_idx(g0, g1):
            def idx_body(g, n):
                off = pl.ds(g * LANES, LANES)
                t = tok_v[off]
                p = base + g * LANES + lax.iota(jnp.int32, LANES)
                b = lax.div(p, L)
                e = jnp.clip(t - 3, 0, N * N - 1)
                idx_v[off] = b * (N * N) + e
                return n
            lax.fori_loop(g0, g1, idx_body, 0)
        compute_idx(0, gpw)

        def gather_of(c, s):
            return (table_hbm.at[idx_v.at[pl.ds(c * CH, CH)]],
                    ebufs[s], gsems[s])

        def write_of(c, s):
            return (ebufs[s],
                    out_hbm.at[pl.ds(base + c * CH, CH)], wsems[s])

        def scan_specials(g0, g1, nfix):
            def scan_body(g, n):
                v = tok_v[pl.ds(g * LANES, LANES)]
                for lane in range(LANES):
                    t = v[lane]
                    @pl.when(t < 3)
                    def _():
                        srow = jnp.clip(t, 0, 2)
                        pltpu.async_copy(
                            sp3.at[pl.ds(srow, 1)],
                            out_hbm.at[pl.ds(base + g * LANES + lane, 1)],
                            fsem)
                    n = n + jnp.where(t < 3, 1, 0)
                return n
            return lax.fori_loop(g0, g1, scan_body, nfix)

        def pipe_body(i, nfix):
            for s in range(NBUF):
                c = i * NBUF + s
                @pl.when(i > 0)
                def _():
                    pltpu.make_async_copy(*write_of(c - NBUF, s)).wait()
                pltpu.async_copy(*gather_of(c, s))
            compute_idx((i + 1) * gpw, jnp.minimum(i + 2, n_iters) * gpw)
            nfix = scan_specials(
                jnp.maximum(i - 1, 0) * gpw, i * gpw, nfix)
            for s in range(NBUF):
                c = i * NBUF + s
                pltpu.make_async_copy(*gather_of(c, s)).wait()
                pltpu.async_copy(*write_of(c, s))
            return nfix
        nfix = lax.fori_loop(0, n_iters, pipe_body, jnp.int32(0))
        for s in range(NBUF):
            c = (n_iters - 1) * NBUF + s
            pltpu.make_async_copy(*write_of(c, s)).wait()
        nfix = scan_specials((n_iters - 1) * gpw, n_iters * gpw, nfix)

        def drain_body(j, z):
            pltpu.make_async_copy(
                sp3.at[pl.ds(0, 1)], out_hbm.at[pl.ds(base, 1)], fsem).wait()
            return z
        lax.fori_loop(0, nfix, drain_body, 0)

    return k


def kernel(tokens, edge_emb, special_token_emb, token_to_edge):
    B, L = tokens.shape
    _, N, _, H = edge_emb.shape
    del token_to_edge
    tok = tokens.reshape(B * L)
    table = edge_emb.reshape(B * N * N, H)
    out = _sc_lookup(B, L, N, H)(tok, table, special_token_emb)
    return out.reshape(B, L, H)
